# baseline (speedup 1.0000x reference)
import jax
import jax.numpy as jnp
from jax.experimental import pallas as pl
from jax.experimental.pallas import tpu as pltpu

EMBED_DIM = 256
NUM_PTM = 26
NUM_PER = 128
NUM_EMB = NUM_PTM * NUM_PER
TEMP = 0.07
NEG = -1e9

BR = 1024
BU = 256
NU = NUM_EMB // BU


def _fused_kernel(q_in_ref, x_ref, emb_ref, emb32_ref, q_ref, idx_ref, out_ref):
    i = pl.program_id(0)
    x = x_ref[...]
    emb = emb_ref[...]
    logits = jax.lax.dot_general(
        x, emb, (((1,), (1,)), ((), ())), preferred_element_type=jnp.float32
    )
    qt = q_in_ref[...]
    RC = 128
    lsel_chunks = []
    for r in range(0, BR, RC):
        qt_c = qt[r:r + RC]
        lsel_c = jnp.zeros((RC, NUM_PER), jnp.float32)
        for t in range(NUM_PTM):
            piece = logits[r:r + RC, t * NUM_PER:(t + 1) * NUM_PER]
            lsel_c = jnp.where(qt_c == t, piece, lsel_c)
        lsel_chunks.append(lsel_c)
    lsel = jnp.concatenate(lsel_chunks, axis=0)
    rowmax = jnp.max(lsel, axis=1, keepdims=True)
    e = jnp.exp(lsel - rowmax)
    s = jnp.sum(e, axis=1, keepdims=True)
    eb = e.astype(jnp.bfloat16)
    ez_chunks = []
    for r in range(0, BR, RC):
        qt_c = qt[r:r + RC]
        eb_c = eb[r:r + RC]
        zero = jnp.zeros_like(eb_c)
        ez_chunks.append(jnp.concatenate(
            [jnp.where(qt_c == t, eb_c, zero) for t in range(NUM_PTM)], axis=1
        ))
    ez = jnp.concatenate(ez_chunks, axis=0)
    q_un = jax.lax.dot_general(
        ez, emb, (((1,), (0,)), ((), ())), preferred_element_type=jnp.float32
    )
    q_ref[...] = q_un / s
    col = jax.lax.broadcasted_iota(jnp.int32, e.shape, 1)
    local = jnp.min(
        jnp.where(lsel == rowmax, col, NUM_PER), axis=1, keepdims=True
    )
    idx_ref[...] = qt * NUM_PER + local

    @pl.when(i == 0)
    def _():
        out_ref[0, 0] = 0.0

    for k in range(2):
        j = 2 * i + k

        @pl.when(j < NU)
        def _():
            emb32 = emb32_ref[...]
            emb_blk = emb32_ref[pl.ds(j * BU, BU), :]
            sim = jax.lax.dot_general(
                emb_blk, emb32, (((1,), (1,)), ((), ())),
                preferred_element_type=jnp.float32,
            )
            ev = jnp.exp(sim * (1.0 / TEMP))
            row_sum = jnp.sum(ev, axis=1, keepdims=True)
            ps = jax.lax.dot_general(
                emb_blk, emb_blk, (((1,), (1,)), ((), ())),
                preferred_element_type=jnp.float32,
            )
            r0 = jax.lax.broadcasted_iota(jnp.int32, ps.shape, 0)
            c0 = jax.lax.broadcasted_iota(jnp.int32, ps.shape, 1)
            pos_mask = (c0 // NUM_PER) == (r0 // NUM_PER)
            pe = jnp.exp(ps * (1.0 / TEMP))
            diag = jnp.sum(jnp.where(r0 == c0, pe, 0.0), axis=1, keepdims=True)
            pos_sum = (
                jnp.sum(jnp.where(pos_mask, pe, 0.0), axis=1, keepdims=True)
                - diag
            )
            sum_exp = row_sum - diag
            out_ref[0, 0] += jnp.sum(jnp.log(pos_sum) - jnp.log(sum_exp))


def kernel(x, Q, embeddings):
    n_rows = x.shape[0]
    grid = n_rows // BR
    qcol = Q.astype(jnp.int32).reshape(-1, 1)

    quantized, idx, total = pl.pallas_call(
        _fused_kernel,
        grid=(grid,),
        in_specs=[
            pl.BlockSpec((BR, 1), lambda i: (i, 0)),
            pl.BlockSpec((BR, EMBED_DIM), lambda i: (i, 0)),
            pl.BlockSpec((NUM_EMB, EMBED_DIM), lambda i: (0, 0)),
            pl.BlockSpec((NUM_EMB, EMBED_DIM), lambda i: (0, 0)),
        ],
        out_specs=[
            pl.BlockSpec((BR, EMBED_DIM), lambda i: (i, 0)),
            pl.BlockSpec((BR, 1), lambda i: (i, 0)),
            pl.BlockSpec(memory_space=pltpu.SMEM),
        ],
        out_shape=[
            jax.ShapeDtypeStruct((n_rows, EMBED_DIM), jnp.float32),
            jax.ShapeDtypeStruct((n_rows, 1), jnp.int32),
            jax.ShapeDtypeStruct((1, 1), jnp.float32),
        ],
    )(qcol, x.astype(jnp.bfloat16), embeddings.astype(jnp.bfloat16), embeddings)

    uniform_loss = -(total[0, 0] / NUM_EMB)
    loss = jnp.float32(0.0)
    return quantized, loss, uniform_loss, idx.reshape(-1)
